# baseline (speedup 1.0000x reference)
import functools

import jax
import jax.numpy as jnp
from jax import lax
from jax.experimental import pallas as pl
from jax.experimental.pallas import tpu as pltpu
from jax.experimental.pallas import tpu_sc as plsc

DIM = 128
SENT_L = 50
PAD_L = 56
SEG = 2 * PAD_L
BATCH = 4096
ALPHA_COEF = 0.1

_NC = 2
_NS = 16
_NW = _NC * _NS
_B_PER_W = BATCH // _NW
_NBUF = 8

_mesh = plsc.VectorSubcoreMesh(core_axis_name="c", subcore_axis_name="s")


def _lanesum(v):
    r = v
    for sh in (8, 4, 2, 1):
        idx = lax.iota(jnp.int32, 16) ^ sh
        r = r + r.at[idx].get(mode="promise_in_bounds")
    return r


@functools.partial(
    pl.kernel,
    mesh=_mesh,
    out_type=(
        jax.ShapeDtypeStruct((BATCH, DIM), jnp.float32),
        jax.ShapeDtypeStruct((BATCH, DIM), jnp.float32),
        jax.ShapeDtypeStruct((BATCH, DIM), jnp.float32),
    ),
    scratch_types=(
        [pltpu.VMEM((_B_PER_W, SEG), jnp.int32)]
        + [pltpu.VMEM((PAD_L, DIM), jnp.float32)] * _NBUF
        + [pltpu.VMEM((DIM,), jnp.float32)]
        + [pltpu.VMEM((_B_PER_W, DIM), jnp.float32)] * 3
        + [pltpu.SemaphoreType.DMA] * _NBUF
    ),
)
def _sc_encode(ids_hbm, embed_hbm, h0_hbm, vp_hbm, vh_hbm, ids_v,
               b0, b1, b2, b3, b4, b5, b6, b7,
               e0_v, st_h0, st_vp, st_vh,
               s0, s1, s2, s3, s4, s5, s6, s7):
    bufs = (b0, b1, b2, b3, b4, b5, b6, b7)
    sems = (s0, s1, s2, s3, s4, s5, s6, s7)

    wid = lax.axis_index("s") * _NC + lax.axis_index("c")
    base = wid * _B_PER_W

    pltpu.sync_copy(ids_hbm.at[pl.ds(base, _B_PER_W)], ids_v)
    pltpu.sync_copy(embed_hbm.at[0], e0_v)

    def _half(i, half, buf, sem):
        idx = ids_v.at[i, pl.ds(half * PAD_L, PAD_L)]
        return pltpu.make_async_copy(embed_hbm.at[idx], buf, sem)

    e0c = tuple(e0_v[pl.ds(16 * c, 16)] for c in range(8))
    lane = lax.iota(jnp.int32, 16)
    front = jnp.minimum(jnp.maximum(8 - lane, 0), 1)

    zero16 = jnp.zeros((16,), jnp.float32)

    def _process(i, bufa, bufb):
        nz = []
        for k in range(7):
            nz.append(jnp.minimum(ids_v[i, pl.ds(16 * k, 16)], 1))
        bound_nz = nz[3] * front
        nonpad_p = _lanesum(nz[0] + nz[1] + nz[2] + bound_nz)
        nonpad_h = _lanesum(nz[3] - bound_nz + nz[4] + nz[5] + nz[6])
        npp = PAD_L - nonpad_p
        nph = PAD_L - nonpad_h

        def abody(r, carry):
            out = []
            for c in range(8):
                out.append(carry[c] + bufa[r, pl.ds(16 * c, 16)])
            for c in range(8):
                out.append(carry[8 + c] + bufb[r, pl.ds(16 * c, 16)])
            return tuple(out)

        accs = lax.fori_loop(0, PAD_L, abody, (zero16,) * 16)

        npp_f = npp.astype(jnp.float32)
        nph_f = nph.astype(jnp.float32)
        denp = jnp.maximum(nonpad_p.astype(jnp.float32), 1.0)
        denh = jnp.maximum(nonpad_h.astype(jnp.float32), 1.0)

        for c in range(8):
            sl = pl.ds(16 * c, 16)
            vp_c = (accs[c] - npp_f * e0c[c]) / denp
            vh_c = (accs[8 + c] - nph_f * e0c[c]) / denh
            h0_c = (vh_c - vp_c) + ALPHA_COEF * (bufb[0, sl] - bufa[0, sl])
            st_h0[i, sl] = h0_c
            st_vp[i, sl] = vp_c
            st_vh[i, sl] = vh_c

    for i in range(_NBUF // 2):
        _half(i, 0, bufs[2 * i], sems[2 * i]).start()
        _half(i, 1, bufs[2 * i + 1], sems[2 * i + 1]).start()

    def gbody(g, carry):
        for j in range(4):
            i = 4 * g + j
            ba, bb = bufs[2 * j], bufs[2 * j + 1]
            sa, sb = sems[2 * j], sems[2 * j + 1]
            _half(i, 0, ba, sa).wait()
            _half(i, 1, bb, sb).wait()
            _process(i, ba, bb)

            @pl.when(g < _B_PER_W // 4 - 1)
            def _():
                _half(i + 4, 0, ba, sa).start()
                _half(i + 4, 1, bb, sb).start()
        return carry

    lax.fori_loop(0, _B_PER_W // 4, gbody, 0)

    pltpu.sync_copy(st_h0, h0_hbm.at[pl.ds(base, _B_PER_W)])
    pltpu.sync_copy(st_vp, vp_hbm.at[pl.ds(base, _B_PER_W)])
    pltpu.sync_copy(st_vh, vh_hbm.at[pl.ds(base, _B_PER_W)])


@jax.jit
def kernel(prem_ids, hyp_ids, embed):
    prem = prem_ids.astype(jnp.int32)
    hyp = hyp_ids.astype(jnp.int32)
    pad = ((0, 0), (0, PAD_L - SENT_L))
    ids = jnp.concatenate([jnp.pad(prem, pad), jnp.pad(hyp, pad)], axis=1)
    h0, v_p, v_h = _sc_encode(ids, embed)
    return (h0, v_p, v_h)


# speedup vs baseline: 2.6630x; 2.6630x over previous
import functools

import jax
import jax.numpy as jnp
from jax import lax
from jax.experimental import pallas as pl
from jax.experimental.pallas import tpu as pltpu
from jax.experimental.pallas import tpu_sc as plsc

DIM = 128
SENT_L = 50
PAD_S = 52
SEG = 2 * PAD_S
BATCH = 4096
ALPHA_COEF = 0.1

_NC = 2
_NS = 16
_NW = _NC * _NS
_B_PER_W = BATCH // _NW
_NBUF = 4

_mesh = plsc.VectorSubcoreMesh(core_axis_name="c", subcore_axis_name="s")


def _lanesum(v):
    r = v
    for sh in (8, 4, 2, 1):
        idx = lax.iota(jnp.int32, 16) ^ sh
        r = r + r.at[idx].get(mode="promise_in_bounds")
    return r


@functools.partial(
    pl.kernel,
    mesh=_mesh,
    out_type=(
        jax.ShapeDtypeStruct((BATCH, DIM), jnp.float32),
        jax.ShapeDtypeStruct((BATCH, DIM), jnp.float32),
        jax.ShapeDtypeStruct((BATCH, DIM), jnp.float32),
    ),
    scratch_types=(
        [pltpu.VMEM((_B_PER_W, SEG), jnp.int32)]
        + [pltpu.VMEM((SEG, DIM), jnp.float32)] * _NBUF
        + [pltpu.VMEM((DIM,), jnp.float32)]
        + [pltpu.VMEM((_B_PER_W, DIM), jnp.float32)] * 3
        + [pltpu.SemaphoreType.DMA] * _NBUF
    ),
)
def _sc_encode(ids_hbm, embed_hbm, h0_hbm, vp_hbm, vh_hbm, ids_v,
               b0, b1, b2, b3,
               e0_v, st_h0, st_vp, st_vh,
               s0, s1, s2, s3):
    bufs = (b0, b1, b2, b3)
    sems = (s0, s1, s2, s3)

    wid = lax.axis_index("s") * _NC + lax.axis_index("c")
    base = wid * _B_PER_W

    pltpu.sync_copy(ids_hbm.at[pl.ds(base, _B_PER_W)], ids_v)
    pltpu.sync_copy(embed_hbm.at[0], e0_v)

    def _row_gather(i, buf, sem):
        return pltpu.make_async_copy(embed_hbm.at[ids_v.at[i]], buf, sem)

    e0c = tuple(e0_v[pl.ds(16 * c, 16)] for c in range(8))
    lane = lax.iota(jnp.int32, 16)
    front4 = jnp.minimum(jnp.maximum(4 - lane, 0), 1)
    back8 = jnp.minimum(jnp.maximum(lane - 7, 0), 1)

    zero16 = jnp.zeros((16,), jnp.float32)

    def _process(i, buf):
        nz = []
        for k in range(6):
            nz.append(jnp.minimum(ids_v[i, pl.ds(16 * k, 16)], 1))
        nz_tail = jnp.minimum(ids_v[i, pl.ds(88, 16)], 1) * back8
        bound_nz = nz[3] * front4
        nonpad_p = _lanesum(nz[0] + nz[1] + nz[2] + bound_nz)
        nonpad_h = _lanesum(nz[3] - bound_nz + nz[4] + nz[5] + nz_tail)
        npp = PAD_S - nonpad_p
        nph = PAD_S - nonpad_h

        def abody(r, carry):
            out = []
            for c in range(8):
                out.append(carry[c] + buf[r, pl.ds(16 * c, 16)])
            for c in range(8):
                out.append(carry[8 + c] + buf[PAD_S + r, pl.ds(16 * c, 16)])
            return tuple(out)

        accs = lax.fori_loop(0, PAD_S, abody, (zero16,) * 16)

        npp_f = npp.astype(jnp.float32)
        nph_f = nph.astype(jnp.float32)
        denp = jnp.maximum(nonpad_p.astype(jnp.float32), 1.0)
        denh = jnp.maximum(nonpad_h.astype(jnp.float32), 1.0)

        for c in range(8):
            sl = pl.ds(16 * c, 16)
            vp_c = (accs[c] - npp_f * e0c[c]) / denp
            vh_c = (accs[8 + c] - nph_f * e0c[c]) / denh
            h0_c = (vh_c - vp_c) + ALPHA_COEF * (
                buf[PAD_S, sl] - buf[0, sl])
            st_h0[i, sl] = h0_c
            st_vp[i, sl] = vp_c
            st_vh[i, sl] = vh_c

    for i in range(_NBUF):
        _row_gather(i, bufs[i], sems[i]).start()

    def gbody(g, carry):
        for j in range(_NBUF):
            i = _NBUF * g + j
            _row_gather(i, bufs[j], sems[j]).wait()
            _process(i, bufs[j])

            @pl.when(g < _B_PER_W // _NBUF - 1)
            def _():
                _row_gather(i + _NBUF, bufs[j], sems[j]).start()
        return carry

    lax.fori_loop(0, _B_PER_W // _NBUF, gbody, 0)

    pltpu.sync_copy(st_h0, h0_hbm.at[pl.ds(base, _B_PER_W)])
    pltpu.sync_copy(st_vp, vp_hbm.at[pl.ds(base, _B_PER_W)])
    pltpu.sync_copy(st_vh, vh_hbm.at[pl.ds(base, _B_PER_W)])


@jax.jit
def kernel(prem_ids, hyp_ids, embed):
    prem = prem_ids.astype(jnp.int32)
    hyp = hyp_ids.astype(jnp.int32)
    pad = ((0, 0), (0, PAD_S - SENT_L))
    ids = jnp.concatenate([jnp.pad(prem, pad), jnp.pad(hyp, pad)], axis=1)
    h0, v_p, v_h = _sc_encode(ids, embed)
    return (h0, v_p, v_h)
